# baseline (speedup 1.0000x reference)
import functools

import jax
import jax.numpy as jnp
from jax import lax
from jax.experimental import pallas as pl
from jax.experimental.pallas import tpu as pltpu
from jax.experimental.pallas import tpu_sc as plsc

NUM_VIEWS = 12
PROJECTION_DIM = 512
BATCH = 4096

_info = plsc.get_sparse_core_info()
NC, NS, L = _info.num_cores, _info.num_subcores, _info.num_lanes
NW = NC * NS

ROWS_PER_W = BATCH // NW
CHUNK = 32
NCHUNK = ROWS_PER_W // CHUNK


def _sc_body(x_hbm, p_hbm, o_hbm, buf, posv):
    wid = lax.axis_index("s") * NC + lax.axis_index("c")
    base = wid * ROWS_PER_W

    def add_row(i, carry):
        for c in range(PROJECTION_DIM // L):
            sl = pl.ds(c * L, L)
            buf[i, sl] = buf[i, sl] + posv[sl]
        return carry

    def chunk_body(t, carry):
        v = t // NCHUNK
        k = t - v * NCHUNK
        r0 = base + k * CHUNK
        pltpu.sync_copy(p_hbm.at[v], posv)
        pltpu.sync_copy(x_hbm.at[v, pl.ds(r0, CHUNK)], buf)
        lax.fori_loop(0, CHUNK, add_row, 0)
        pltpu.sync_copy(buf, o_hbm.at[v, pl.ds(r0, CHUNK)])
        return carry

    lax.fori_loop(0, NUM_VIEWS * NCHUNK, chunk_body, 0)


def _sc_call(xt, pos_table):
    mesh = plsc.VectorSubcoreMesh(core_axis_name="c", subcore_axis_name="s")
    return pl.kernel(
        _sc_body,
        mesh=mesh,
        out_type=jax.ShapeDtypeStruct((NUM_VIEWS, BATCH, PROJECTION_DIM), jnp.float32),
        scratch_types=[
            pltpu.VMEM((CHUNK, PROJECTION_DIM), jnp.float32),
            pltpu.VMEM((PROJECTION_DIM,), jnp.float32),
        ],
    )(xt, pos_table)


def kernel(onedimage, pos_table):
    xt = jnp.transpose(onedimage, (1, 0, 2))
    out_t = _sc_call(xt, pos_table)
    return jnp.transpose(out_t, (1, 0, 2))


# speedup vs baseline: 1.4503x; 1.4503x over previous
import jax
import jax.numpy as jnp
from jax import lax
from jax.experimental import pallas as pl
from jax.experimental.pallas import tpu as pltpu
from jax.experimental.pallas import tpu_sc as plsc

NUM_VIEWS = 12
PROJECTION_DIM = 512
BATCH = 4096

_info = plsc.get_sparse_core_info()
NC, NS, L = _info.num_cores, _info.num_subcores, _info.num_lanes
NW = NC * NS

ROWS_PER_W = BATCH // NW
CHUNK = 16
NCHUNK = ROWS_PER_W // CHUNK
NBUF = 4
TOTAL = NUM_VIEWS * NCHUNK
NGROUP = TOTAL // NBUF


def _sc_body(x_hbm, p_hbm, o_hbm, ibuf, obuf, posall, isem, osem):
    wid = lax.axis_index("s") * NC + lax.axis_index("c")
    base = wid * ROWS_PER_W

    pltpu.sync_copy(p_hbm, posall)

    def src_slice(t):
        v = t // NCHUNK
        r0 = base + (t - v * NCHUNK) * CHUNK
        return v, r0

    def in_copy(t, b):
        v, r0 = src_slice(t)
        return pltpu.make_async_copy(
            x_hbm.at[v, pl.ds(r0, CHUNK)], ibuf.at[b], isem.at[b]
        )

    def out_copy(t, b):
        v, r0 = src_slice(t)
        return pltpu.make_async_copy(
            obuf.at[b], o_hbm.at[v, pl.ds(r0, CHUNK)], osem.at[b]
        )

    for b in range(NBUF):
        in_copy(b, b).start()

    def group(g, carry):
        for b in range(NBUF):
            t = g * NBUF + b
            v = t // NCHUNK
            in_copy(t, b).wait()

            @pl.when(t >= NBUF)
            def _():
                out_copy(t - NBUF, b).wait()

            def add_row(i, c2):
                for c in range(PROJECTION_DIM // L):
                    sl = pl.ds(c * L, L)
                    obuf[b, i, sl] = ibuf[b, i, sl] + posall[v, sl]
                return c2

            lax.fori_loop(0, CHUNK, add_row, 0)
            out_copy(t, b).start()

            @pl.when(t + NBUF < TOTAL)
            def _():
                in_copy(t + NBUF, b).start()
        return carry

    lax.fori_loop(0, NGROUP, group, 0)

    for b in range(NBUF):
        out_copy(TOTAL - NBUF + b, b).wait()


def _sc_call(xt, pos_table):
    mesh = plsc.VectorSubcoreMesh(core_axis_name="c", subcore_axis_name="s")
    return pl.kernel(
        _sc_body,
        mesh=mesh,
        out_type=jax.ShapeDtypeStruct((NUM_VIEWS, BATCH, PROJECTION_DIM), jnp.float32),
        scratch_types=[
            pltpu.VMEM((NBUF, CHUNK, PROJECTION_DIM), jnp.float32),
            pltpu.VMEM((NBUF, CHUNK, PROJECTION_DIM), jnp.float32),
            pltpu.VMEM((NUM_VIEWS, PROJECTION_DIM), jnp.float32),
            pltpu.SemaphoreType.DMA((NBUF,)),
            pltpu.SemaphoreType.DMA((NBUF,)),
        ],
    )(xt, pos_table)


def kernel(onedimage, pos_table):
    xt = jnp.transpose(onedimage, (1, 0, 2))
    out_t = _sc_call(xt, pos_table)
    return jnp.transpose(out_t, (1, 0, 2))


# speedup vs baseline: 1.4822x; 1.0220x over previous
import jax
import jax.numpy as jnp
from jax import lax
from jax.experimental import pallas as pl
from jax.experimental.pallas import tpu as pltpu
from jax.experimental.pallas import tpu_sc as plsc

NUM_VIEWS = 12
PROJECTION_DIM = 512
BATCH = 4096

_info = plsc.get_sparse_core_info()
NC, NS, L = _info.num_cores, _info.num_subcores, _info.num_lanes
NW = NC * NS

ROWS_PER_W = BATCH // NW
CHUNK = 32
NCHUNK = ROWS_PER_W // CHUNK
NBUF = 3
TOTAL = NUM_VIEWS * NCHUNK
NGROUP = TOTAL // NBUF


def _sc_body(x_hbm, p_hbm, o_hbm, ibuf, obuf, posall, isem, osem):
    wid = lax.axis_index("s") * NC + lax.axis_index("c")
    base = wid * ROWS_PER_W

    pltpu.sync_copy(p_hbm, posall)

    def src_slice(t):
        v = t // NCHUNK
        r0 = base + (t - v * NCHUNK) * CHUNK
        return v, r0

    def in_copy(t, b):
        v, r0 = src_slice(t)
        return pltpu.make_async_copy(
            x_hbm.at[v, pl.ds(r0, CHUNK)], ibuf.at[b], isem.at[b]
        )

    def out_copy(t, b):
        v, r0 = src_slice(t)
        return pltpu.make_async_copy(
            obuf.at[b], o_hbm.at[v, pl.ds(r0, CHUNK)], osem.at[b]
        )

    for b in range(NBUF):
        in_copy(b, b).start()

    def group(g, carry):
        for b in range(NBUF):
            t = g * NBUF + b
            v = t // NCHUNK
            in_copy(t, b).wait()

            @pl.when(t >= NBUF)
            def _():
                out_copy(t - NBUF, b).wait()

            def add_row(i, c2):
                for c in range(PROJECTION_DIM // L):
                    sl = pl.ds(c * L, L)
                    obuf[b, i, sl] = ibuf[b, i, sl] + posall[v, sl]
                return c2

            lax.fori_loop(0, CHUNK, add_row, 0)
            out_copy(t, b).start()

            @pl.when(t + NBUF < TOTAL)
            def _():
                in_copy(t + NBUF, b).start()
        return carry

    lax.fori_loop(0, NGROUP, group, 0)

    for b in range(NBUF):
        out_copy(TOTAL - NBUF + b, b).wait()


def _sc_call(xt, pos_table):
    mesh = plsc.VectorSubcoreMesh(core_axis_name="c", subcore_axis_name="s")
    return pl.kernel(
        _sc_body,
        mesh=mesh,
        out_type=jax.ShapeDtypeStruct((NUM_VIEWS, BATCH, PROJECTION_DIM), jnp.float32),
        scratch_types=[
            pltpu.VMEM((NBUF, CHUNK, PROJECTION_DIM), jnp.float32),
            pltpu.VMEM((NBUF, CHUNK, PROJECTION_DIM), jnp.float32),
            pltpu.VMEM((NUM_VIEWS, PROJECTION_DIM), jnp.float32),
            pltpu.SemaphoreType.DMA((NBUF,)),
            pltpu.SemaphoreType.DMA((NBUF,)),
        ],
    )(xt, pos_table)


def kernel(onedimage, pos_table):
    xt = jnp.transpose(onedimage, (1, 0, 2))
    out_t = _sc_call(xt, pos_table)
    return jnp.transpose(out_t, (1, 0, 2))


# speedup vs baseline: 2.6261x; 1.7718x over previous
import jax
import jax.numpy as jnp
from jax import lax
from jax.experimental import pallas as pl
from jax.experimental.pallas import tpu as pltpu
from jax.experimental.pallas import tpu_sc as plsc

NUM_VIEWS = 12
PROJECTION_DIM = 512
BATCH = 4096

V_SC = 2
V_TC = NUM_VIEWS - V_SC

_info = plsc.get_sparse_core_info()
NC, NS, L = _info.num_cores, _info.num_subcores, _info.num_lanes
NW = NC * NS

ROWS_PER_W = BATCH // NW
CHUNK = 16
NCHUNK = ROWS_PER_W // CHUNK
NBUF = 4
TOTAL = V_SC * NCHUNK
NGROUP = TOTAL // NBUF


def _tc_body(x_ref, p_ref, o_ref):
    o_ref[...] = x_ref[...] + p_ref[...]


def _sc_body(x_hbm, p_hbm, o_hbm, ibuf, obuf, posall, isem, osem):
    wid = lax.axis_index("s") * NC + lax.axis_index("c")
    base = wid * ROWS_PER_W

    pltpu.sync_copy(p_hbm, posall)

    def src_slice(t):
        vo = t // NCHUNK
        r0 = base + (t - vo * NCHUNK) * CHUNK
        return vo, r0

    def in_copy(t, b):
        vo, r0 = src_slice(t)
        return pltpu.make_async_copy(
            x_hbm.at[V_TC + vo, pl.ds(r0, CHUNK)], ibuf.at[b], isem.at[b]
        )

    def out_copy(t, b):
        vo, r0 = src_slice(t)
        return pltpu.make_async_copy(
            obuf.at[b], o_hbm.at[vo, pl.ds(r0, CHUNK)], osem.at[b]
        )

    for b in range(NBUF):
        in_copy(b, b).start()

    def group(g, carry):
        for b in range(NBUF):
            t = g * NBUF + b
            vo = t // NCHUNK
            in_copy(t, b).wait()

            @pl.when(t >= NBUF)
            def _():
                out_copy(t - NBUF, b).wait()

            def add_row(i, c2):
                for c in range(PROJECTION_DIM // L):
                    sl = pl.ds(c * L, L)
                    obuf[b, i, sl] = ibuf[b, i, sl] + posall[V_TC + vo, sl]
                return c2

            lax.fori_loop(0, CHUNK, add_row, 0)
            out_copy(t, b).start()

            @pl.when(t + NBUF < TOTAL)
            def _():
                in_copy(t + NBUF, b).start()
        return carry

    lax.fori_loop(0, NGROUP, group, 0)

    for b in range(NBUF):
        out_copy(TOTAL - NBUF + b, b).wait()


def _sc_call(xt, pos_table):
    mesh = plsc.VectorSubcoreMesh(core_axis_name="c", subcore_axis_name="s")
    return pl.kernel(
        _sc_body,
        mesh=mesh,
        out_type=jax.ShapeDtypeStruct((V_SC, BATCH, PROJECTION_DIM), jnp.float32),
        scratch_types=[
            pltpu.VMEM((NBUF, CHUNK, PROJECTION_DIM), jnp.float32),
            pltpu.VMEM((NBUF, CHUNK, PROJECTION_DIM), jnp.float32),
            pltpu.VMEM((NUM_VIEWS, PROJECTION_DIM), jnp.float32),
            pltpu.SemaphoreType.DMA((NBUF,)),
            pltpu.SemaphoreType.DMA((NBUF,)),
        ],
    )(xt, pos_table)


def _tc_call(xt, pos_table):
    p3 = pos_table.reshape(NUM_VIEWS, 1, PROJECTION_DIM)
    return pl.pallas_call(
        _tc_body,
        grid=(V_TC,),
        in_specs=[
            pl.BlockSpec((1, BATCH, PROJECTION_DIM), lambda v: (v, 0, 0)),
            pl.BlockSpec((1, 1, PROJECTION_DIM), lambda v: (v, 0, 0)),
        ],
        out_specs=pl.BlockSpec((1, BATCH, PROJECTION_DIM), lambda v: (v, 0, 0)),
        out_shape=jax.ShapeDtypeStruct((V_TC, BATCH, PROJECTION_DIM), jnp.float32),
    )(xt, p3)


def kernel(onedimage, pos_table):
    xt = jnp.transpose(onedimage, (1, 0, 2))
    sc_out = _sc_call(xt, pos_table)
    tc_out = _tc_call(xt, pos_table)
    out_t = jnp.concatenate([tc_out, sc_out], axis=0)
    return jnp.transpose(out_t, (1, 0, 2))


# speedup vs baseline: 3.2297x; 1.2298x over previous
import jax
import jax.numpy as jnp
from jax import lax
from jax.experimental import pallas as pl
from jax.experimental.pallas import tpu as pltpu
from jax.experimental.pallas import tpu_sc as plsc

NUM_VIEWS = 12
PROJECTION_DIM = 512
BATCH = 4096

V_SC = 2
V_TC = NUM_VIEWS - V_SC

_info = plsc.get_sparse_core_info()
NC, NS, L = _info.num_cores, _info.num_subcores, _info.num_lanes
NW = NC * NS

ROWS_PER_W = BATCH // NW
CHUNK = 16
NCHUNK = ROWS_PER_W // CHUNK
NBUF = 4
TOTAL = V_SC * NCHUNK
NGROUP = TOTAL // NBUF


def _tc_body(x_ref, p_ref, _sc_ref, o_ref):
    o_ref[...] = x_ref[...] + p_ref[...]


def _sc_body(x_hbm, p_hbm, o_hbm, ibuf, obuf, posall, isem, osem):
    wid = lax.axis_index("s") * NC + lax.axis_index("c")
    base = wid * ROWS_PER_W

    pltpu.sync_copy(p_hbm, posall)

    def src_slice(t):
        vo = t // NCHUNK
        r0 = base + (t - vo * NCHUNK) * CHUNK
        return vo, r0

    def in_copy(t, b):
        vo, r0 = src_slice(t)
        return pltpu.make_async_copy(
            x_hbm.at[V_TC + vo, pl.ds(r0, CHUNK)], ibuf.at[b], isem.at[b]
        )

    def out_copy(t, b):
        vo, r0 = src_slice(t)
        return pltpu.make_async_copy(
            obuf.at[b], o_hbm.at[V_TC + vo, pl.ds(r0, CHUNK)], osem.at[b]
        )

    for b in range(NBUF):
        in_copy(b, b).start()

    def group(g, carry):
        for b in range(NBUF):
            t = g * NBUF + b
            vo = t // NCHUNK
            in_copy(t, b).wait()

            @pl.when(t >= NBUF)
            def _():
                out_copy(t - NBUF, b).wait()

            def add_row(i, c2):
                for c in range(PROJECTION_DIM // L):
                    sl = pl.ds(c * L, L)
                    obuf[b, i, sl] = ibuf[b, i, sl] + posall[V_TC + vo, sl]
                return c2

            lax.fori_loop(0, CHUNK, add_row, 0)
            out_copy(t, b).start()

            @pl.when(t + NBUF < TOTAL)
            def _():
                in_copy(t + NBUF, b).start()
        return carry

    lax.fori_loop(0, NGROUP, group, 0)

    for b in range(NBUF):
        out_copy(TOTAL - NBUF + b, b).wait()


def _sc_call(xt, pos_table):
    mesh = plsc.VectorSubcoreMesh(core_axis_name="c", subcore_axis_name="s")
    return pl.kernel(
        _sc_body,
        mesh=mesh,
        out_type=jax.ShapeDtypeStruct((NUM_VIEWS, BATCH, PROJECTION_DIM), jnp.float32),
        scratch_types=[
            pltpu.VMEM((NBUF, CHUNK, PROJECTION_DIM), jnp.float32),
            pltpu.VMEM((NBUF, CHUNK, PROJECTION_DIM), jnp.float32),
            pltpu.VMEM((NUM_VIEWS, PROJECTION_DIM), jnp.float32),
            pltpu.SemaphoreType.DMA((NBUF,)),
            pltpu.SemaphoreType.DMA((NBUF,)),
        ],
    )(xt, pos_table)


def _tc_call(xt, pos_table, sc_full):
    p3 = pos_table.reshape(NUM_VIEWS, 1, PROJECTION_DIM)
    return pl.pallas_call(
        _tc_body,
        grid=(V_TC,),
        in_specs=[
            pl.BlockSpec((1, BATCH, PROJECTION_DIM), lambda v: (v, 0, 0)),
            pl.BlockSpec((1, 1, PROJECTION_DIM), lambda v: (v, 0, 0)),
            pl.BlockSpec(memory_space=pl.ANY),
        ],
        out_specs=pl.BlockSpec((1, BATCH, PROJECTION_DIM), lambda v: (v, 0, 0)),
        out_shape=jax.ShapeDtypeStruct((NUM_VIEWS, BATCH, PROJECTION_DIM), jnp.float32),
        input_output_aliases={2: 0},
    )(xt, p3, sc_full)


def kernel(onedimage, pos_table):
    xt = jnp.transpose(onedimage, (1, 0, 2))
    sc_full = _sc_call(xt, pos_table)
    out_t = _tc_call(xt, pos_table, sc_full)
    return jnp.transpose(out_t, (1, 0, 2))


# speedup vs baseline: 3.6781x; 1.1388x over previous
import jax
import jax.numpy as jnp
from jax import lax
from jax.experimental import pallas as pl
from jax.experimental.pallas import tpu as pltpu
from jax.experimental.pallas import tpu_sc as plsc

NUM_VIEWS = 12
PROJECTION_DIM = 512
BATCH = 4096

V_SC = 1
V_TC = NUM_VIEWS - V_SC

_info = plsc.get_sparse_core_info()
NC, NS, L = _info.num_cores, _info.num_subcores, _info.num_lanes
NW = NC * NS

ROWS_PER_W = BATCH // NW
CHUNK = 16
NCHUNK = ROWS_PER_W // CHUNK
NBUF = 4
TOTAL = V_SC * NCHUNK
NGROUP = TOTAL // NBUF


def _tc_body(x_ref, p_ref, _sc_ref, o_ref):
    o_ref[...] = x_ref[...] + p_ref[...]


def _sc_body(x_hbm, p_hbm, o_hbm, ibuf, obuf, posall, isem, osem):
    wid = lax.axis_index("s") * NC + lax.axis_index("c")
    base = wid * ROWS_PER_W

    pltpu.sync_copy(p_hbm, posall)

    def src_slice(t):
        vo = t // NCHUNK
        r0 = base + (t - vo * NCHUNK) * CHUNK
        return vo, r0

    def in_copy(t, b):
        vo, r0 = src_slice(t)
        return pltpu.make_async_copy(
            x_hbm.at[V_TC + vo, pl.ds(r0, CHUNK)], ibuf.at[b], isem.at[b]
        )

    def out_copy(t, b):
        vo, r0 = src_slice(t)
        return pltpu.make_async_copy(
            obuf.at[b], o_hbm.at[V_TC + vo, pl.ds(r0, CHUNK)], osem.at[b]
        )

    for b in range(NBUF):
        in_copy(b, b).start()

    def group(g, carry):
        for b in range(NBUF):
            t = g * NBUF + b
            vo = t // NCHUNK
            in_copy(t, b).wait()

            @pl.when(t >= NBUF)
            def _():
                out_copy(t - NBUF, b).wait()

            def add_row(i, c2):
                for c in range(PROJECTION_DIM // L):
                    sl = pl.ds(c * L, L)
                    obuf[b, i, sl] = ibuf[b, i, sl] + posall[V_TC + vo, sl]
                return c2

            lax.fori_loop(0, CHUNK, add_row, 0)
            out_copy(t, b).start()

            @pl.when(t + NBUF < TOTAL)
            def _():
                in_copy(t + NBUF, b).start()
        return carry

    lax.fori_loop(0, NGROUP, group, 0)

    for b in range(NBUF):
        out_copy(TOTAL - NBUF + b, b).wait()


def _sc_call(xt, pos_table):
    mesh = plsc.VectorSubcoreMesh(core_axis_name="c", subcore_axis_name="s")
    return pl.kernel(
        _sc_body,
        mesh=mesh,
        out_type=jax.ShapeDtypeStruct((NUM_VIEWS, BATCH, PROJECTION_DIM), jnp.float32),
        scratch_types=[
            pltpu.VMEM((NBUF, CHUNK, PROJECTION_DIM), jnp.float32),
            pltpu.VMEM((NBUF, CHUNK, PROJECTION_DIM), jnp.float32),
            pltpu.VMEM((NUM_VIEWS, PROJECTION_DIM), jnp.float32),
            pltpu.SemaphoreType.DMA((NBUF,)),
            pltpu.SemaphoreType.DMA((NBUF,)),
        ],
    )(xt, pos_table)


def _tc_call(xt, pos_table, sc_full):
    p3 = pos_table.reshape(NUM_VIEWS, 1, PROJECTION_DIM)
    return pl.pallas_call(
        _tc_body,
        grid=(V_TC,),
        in_specs=[
            pl.BlockSpec((1, BATCH, PROJECTION_DIM), lambda v: (v, 0, 0)),
            pl.BlockSpec((1, 1, PROJECTION_DIM), lambda v: (v, 0, 0)),
            pl.BlockSpec(memory_space=pl.ANY),
        ],
        out_specs=pl.BlockSpec((1, BATCH, PROJECTION_DIM), lambda v: (v, 0, 0)),
        out_shape=jax.ShapeDtypeStruct((NUM_VIEWS, BATCH, PROJECTION_DIM), jnp.float32),
        input_output_aliases={2: 0},
    )(xt, p3, sc_full)


def kernel(onedimage, pos_table):
    xt = jnp.transpose(onedimage, (1, 0, 2))
    sc_full = _sc_call(xt, pos_table)
    out_t = _tc_call(xt, pos_table, sc_full)
    return jnp.transpose(out_t, (1, 0, 2))
